# baseline (speedup 1.0000x reference)
import functools

import jax
import jax.numpy as jnp
from jax import lax
from jax.experimental import pallas as pl
from jax.experimental.pallas import tpu as pltpu
from jax.experimental.pallas import tpu_sc as plsc

VOCAB = 1000000
EMBED = 64
BATCH = 16384

NC = 2
NS = 16
L = 16
NW = NC * NS
BPW = BATCH // NW
CHUNK = 128
NCHUNK = BPW // CHUNK


def _sc_body(center_hbm, context_hbm, inA_hbm, outB_hbm, part_hbm,
             cidx, xidx, arows, brows, part_v, sem):
  wid = lax.axis_index("s") * NC + lax.axis_index("c")
  base = wid * BPW

  pltpu.sync_copy(center_hbm.at[pl.ds(base, BPW)], cidx)
  pltpu.sync_copy(context_hbm.at[pl.ds(base, BPW)], xidx)

  copies = []
  for c in range(NCHUNK):
    sl = pl.ds(c * CHUNK, CHUNK)
    copies.append(pltpu.async_copy(inA_hbm.at[cidx.at[sl]], arows.at[sl], sem))
    copies.append(pltpu.async_copy(outB_hbm.at[xidx.at[sl]], brows.at[sl], sem))
  for cp in copies:
    cp.wait()

  def row_body(r, _):
    acc = arows[r, pl.ds(0, L)] * brows[r, pl.ds(0, L)]
    for k in range(1, EMBED // L):
      acc = acc + arows[r, pl.ds(k * L, L)] * brows[r, pl.ds(k * L, L)]
    part_v[r, :] = acc
    return _

  lax.fori_loop(0, BPW, row_body, None)

  pltpu.sync_copy(part_v, part_hbm.at[pl.ds(base, BPW)])


_sc_partials = pl.kernel(
    _sc_body,
    out_type=jax.ShapeDtypeStruct((BATCH, L), jnp.float32),
    mesh=plsc.VectorSubcoreMesh(core_axis_name="c", subcore_axis_name="s",
                                num_cores=NC, num_subcores=NS),
    scratch_types=[
        pltpu.VMEM((BPW,), jnp.int32),
        pltpu.VMEM((BPW,), jnp.int32),
        pltpu.VMEM((BPW, EMBED), jnp.float32),
        pltpu.VMEM((BPW, EMBED), jnp.float32),
        pltpu.VMEM((BPW, L), jnp.float32),
        pltpu.SemaphoreType.DMA,
    ],
    compiler_params=pltpu.CompilerParams(use_tc_tiling_on_sc=False),
)


def _tc_body(part_ref, o_ref):
  s = jnp.sum(part_ref[...], axis=1)
  lp = jnp.minimum(s, 0.0) - jnp.log1p(jnp.exp(-jnp.abs(s)))
  o_ref[0, 0] = -jnp.sum(lp) / BATCH


def kernel(center, context, input_embed, output_embed):
  part = _sc_partials(center.astype(jnp.int32), context.astype(jnp.int32),
                      input_embed, output_embed)
  out = pl.pallas_call(
      _tc_body,
      out_shape=jax.ShapeDtypeStruct((1, 1), jnp.float32),
      out_specs=pl.BlockSpec(memory_space=pltpu.SMEM),
  )(part)
  return out[0, 0]
